# baseline (speedup 1.0000x reference)
import jax
import jax.numpy as jnp
from jax.experimental import pallas as pl
from jax.experimental.pallas import tpu as pltpu

_TOP_N = 2
_REG_WEIGHT = 0.05
_BC = 4096

_TRANS_DIMS = (((0,), (0,)), ((), ()))


_CHUNK = 256


def _cox_kernel(rand_ref, y_ref, len_ref, ev_ref, out_ref):
    bc, nsq = rand_ref.shape
    n = y_ref.shape[0]
    ck = min(_CHUNK, bc)

    eye = (jax.lax.broadcasted_iota(jnp.int32, (ck, ck), 0)
           == jax.lax.broadcasted_iota(jnp.int32, (ck, ck), 1)
           ).astype(jnp.float32)

    def tr(x):
        return jnp.transpose(x)

    sel = (jax.lax.broadcasted_iota(jnp.int32, (n, n * n), 1) // n
           == jax.lax.broadcasted_iota(jnp.int32, (n, n * n), 0)
           ).astype(jnp.float32)

    for s in range(bc // ck):
        lo = s * ck
        y_t = y_ref[:, pl.ds(lo, ck)]
        len_t = len_ref[:, pl.ds(lo, ck)]
        ev_t = ev_ref[:, pl.ds(lo, ck)]
        rand3 = tr(rand_ref[pl.ds(lo, ck), :]).reshape(n, n, ck)

        maxy = jnp.max(y_t, axis=0, keepdims=True)
        eexp = jnp.exp(y_t - maxy)

        len_j = len_t.reshape(1, n, ck)
        len_i = len_t.reshape(n, 1, ck)
        evt_i = ev_t.reshape(n, 1, ck)

        pair = jnp.where(len_j - len_i > 0.0, evt_i, 0.0)

        p = pair * (1.0 + rand3)
        m1 = jnp.max(p, axis=1, keepdims=True)
        keep = p == m1
        for t in range(1, _TOP_N):
            m1 = jnp.max(jnp.where(keep, -1.0, p), axis=1, keepdims=True)
            keep = keep | (p == m1)
        kept = (keep & (p > 0.0)).astype(jnp.float32)

        kflat = kept.reshape(n * n, ck)
        wflat = (kept * eexp.reshape(1, n, ck)).reshape(n * n, ck)
        row_sum = jnp.dot(sel, kflat, preferred_element_type=jnp.float32,
                          precision=jax.lax.Precision.HIGHEST)
        exp_dot = jnp.dot(sel, wflat, preferred_element_type=jnp.float32,
                          precision=jax.lax.Precision.HIGHEST)

        valid = row_sum != 0.0
        valid_f = valid.astype(jnp.float32)
        exp_sum = jnp.where(valid, exp_dot + valid_f * eexp, 1.0)
        per_i = jnp.where(valid, maxy - y_t + jnp.log(exp_sum), 0.0)

        loss_main = jnp.sum(per_i, axis=0, keepdims=True)
        cols_sum = jnp.sum(kept, axis=0) + valid_f
        reg = jnp.sum(jnp.abs(cols_sum * y_t), axis=0, keepdims=True)
        out_ref[0:1, pl.ds(lo, ck)] = loss_main + _REG_WEIGHT * reg


@jax.jit
def _cox_batched(y_pred, length, event, rand_mat):
    B, n = rand_mat.shape[0], rand_mat.shape[-1]
    y = y_pred.reshape(B, n).astype(jnp.float32).T
    ln = length.reshape(B, n).astype(jnp.float32).T
    ev = event.reshape(B, n).astype(jnp.float32).T
    rnd = rand_mat.astype(jnp.float32).reshape(B, n * n)

    bc = min(_BC, B)
    while B % bc:
        bc -= 1
    c = B // bc

    out = pl.pallas_call(
        _cox_kernel,
        out_shape=jax.ShapeDtypeStruct((1, B), jnp.float32),
        grid=(c,),
        in_specs=[
            pl.BlockSpec((bc, n * n), lambda i: (i, 0)),
            pl.BlockSpec((n, bc), lambda i: (0, i)),
            pl.BlockSpec((n, bc), lambda i: (0, i)),
            pl.BlockSpec((n, bc), lambda i: (0, i)),
        ],
        out_specs=pl.BlockSpec((1, bc), lambda i: (0, i)),
        compiler_params=pltpu.CompilerParams(
            dimension_semantics=("parallel",)),
    )(rnd, y, ln, ev)
    return out.reshape(B)


def kernel(y_pred, length, event, rand_mat):
    return _cox_batched(y_pred, length, event, rand_mat)


# speedup vs baseline: 1.0143x; 1.0143x over previous
import jax
import jax.numpy as jnp
from jax.experimental import pallas as pl
from jax.experimental.pallas import tpu as pltpu

_TOP_N = 2
_REG_WEIGHT = 0.05
_BC = 4096
_CHUNK = 128


def _cox_kernel(rand_ref, y_ref, len_ref, ev_ref, out_ref):
    bc = rand_ref.shape[0]
    n = y_ref.shape[0]
    ck = min(_CHUNK, bc)

    sel = (jax.lax.broadcasted_iota(jnp.int32, (n, n * n), 1) // n
           == jax.lax.broadcasted_iota(jnp.int32, (n, n * n), 0)
           ).astype(jnp.float32)

    for s in range(bc // ck):
        lo = s * ck
        y_t = y_ref[:, pl.ds(lo, ck)]
        len_t = len_ref[:, pl.ds(lo, ck)]
        ev_t = ev_ref[:, pl.ds(lo, ck)]
        rand3 = jnp.transpose(rand_ref[pl.ds(lo, ck), :]).reshape(n, n, ck)

        maxy = jnp.max(y_t, axis=0, keepdims=True)
        eexp = jnp.exp(y_t - maxy)

        len_j = len_t.reshape(1, n, ck)
        len_i = len_t.reshape(n, 1, ck)
        evt_i = ev_t.reshape(n, 1, ck)

        pair = jnp.where(len_j - len_i > 0.0, evt_i, 0.0)

        p = pair * (1.0 + rand3)
        m1 = jnp.max(p, axis=1, keepdims=True)
        keep = p == m1
        for _ in range(1, _TOP_N):
            m1 = jnp.max(jnp.where(keep, -1.0, p), axis=1, keepdims=True)
            keep = keep | (p == m1)
        kept = (keep & (p > 0.0)).astype(jnp.float32)

        kflat = kept.reshape(n * n, ck)
        wflat = (kept * eexp.reshape(1, n, ck)).reshape(n * n, ck)
        row_sum = jnp.dot(sel, kflat, preferred_element_type=jnp.float32,
                          precision=jax.lax.Precision.HIGHEST)
        exp_dot = jnp.dot(sel, wflat, preferred_element_type=jnp.float32,
                          precision=jax.lax.Precision.HIGHEST)

        valid = row_sum != 0.0
        valid_f = valid.astype(jnp.float32)
        exp_sum = jnp.where(valid, exp_dot + valid_f * eexp, 1.0)
        per_i = jnp.where(valid, maxy - y_t + jnp.log(exp_sum), 0.0)

        loss_main = jnp.sum(per_i, axis=0, keepdims=True)
        cols_sum = jnp.sum(kept, axis=0) + valid_f
        reg = jnp.sum(jnp.abs(cols_sum * y_t), axis=0, keepdims=True)
        out_ref[0:1, pl.ds(lo, ck)] = loss_main + _REG_WEIGHT * reg


@jax.jit
def _cox_batched(y_pred, length, event, rand_mat):
    B, n = rand_mat.shape[0], rand_mat.shape[-1]
    y = y_pred.reshape(B, n).astype(jnp.float32).T
    ln = length.reshape(B, n).astype(jnp.float32).T
    ev = event.reshape(B, n).astype(jnp.float32).T
    rnd = rand_mat.astype(jnp.float32).reshape(B, n * n)

    bc = min(_BC, B)
    while B % bc:
        bc -= 1
    c = B // bc

    out = pl.pallas_call(
        _cox_kernel,
        out_shape=jax.ShapeDtypeStruct((1, B), jnp.float32),
        grid=(c,),
        in_specs=[
            pl.BlockSpec((bc, n * n), lambda i: (i, 0)),
            pl.BlockSpec((n, bc), lambda i: (0, i)),
            pl.BlockSpec((n, bc), lambda i: (0, i)),
            pl.BlockSpec((n, bc), lambda i: (0, i)),
        ],
        out_specs=pl.BlockSpec((1, bc), lambda i: (0, i)),
        compiler_params=pltpu.CompilerParams(
            dimension_semantics=("parallel",)),
    )(rnd, y, ln, ev)
    return out.reshape(B)


def kernel(y_pred, length, event, rand_mat):
    return _cox_batched(y_pred, length, event, rand_mat)
